# Initial kernel scaffold; baseline (speedup 1.0000x reference)
import jax
import jax.numpy as jnp
from jax.experimental import pallas as pl


def kernel(x, table, bias):
    raise NotImplementedError("write your pallas kernel here")


# speedup vs baseline: 1.0724x; 1.0724x over previous
import functools

import jax
import jax.numpy as jnp
from jax import lax
from jax.experimental import pallas as pl
from jax.experimental.pallas import tpu as pltpu
from jax.experimental.pallas import tpu_sc as plsc

_B, _F = 16384, 26
_OFF = 100000
_NC, _NS, _L = 2, 16, 16
_NW = _NC * _NS
_BPW = _B // _NW
_IPW = _BPW * _F
_NCHUNK = _IPW // _L
_SLEN = 128
_NSTREAM = _IPW // _SLEN
_DEPTH = 6

_mesh = plsc.VectorSubcoreMesh(core_axis_name="c", subcore_axis_name="s")


@functools.partial(
    pl.kernel,
    out_type=jax.ShapeDtypeStruct((_B,), jnp.float32),
    mesh=_mesh,
    compiler_params=pltpu.CompilerParams(needs_layout_passes=False),
    scratch_types=[
        pltpu.VMEM((_IPW,), jnp.int32),
        pltpu.VMEM((_IPW,), jnp.float32),
        pltpu.VMEM((_BPW,), jnp.float32),
        pltpu.VMEM((_L,), jnp.float32),
        pltpu.SemaphoreType.DMA,
    ],
)
def _lr_kernel(x_hbm, table_hbm, bias_hbm, out_hbm, idx_v, vals_v, out_v,
               bias_v, sem):
    wid = lax.axis_index("s") * _NC + lax.axis_index("c")
    base = wid * _IPW

    pltpu.sync_copy(x_hbm.at[pl.ds(base, _IPW)], idx_v)
    pltpu.sync_copy(bias_hbm, bias_v)

    iota = lax.iota(jnp.int32, _L)

    def off_body(c, _):
        sl = pl.ds(c * _L, _L)
        f = lax.rem(c * _L + iota, _F)
        idx_v[sl] = idx_v[sl] + f * _OFF
        return _

    lax.fori_loop(0, _NCHUNK, off_body, None)

    def fire(r):
        sl = pl.ds(r * _SLEN, _SLEN)
        pltpu.async_copy(table_hbm.at[idx_v.at[sl]], vals_v.at[sl], sem)

    def drain(r):
        sl = pl.ds(r * _SLEN, _SLEN)
        pltpu.make_async_copy(table_hbm.at[idx_v.at[sl]], vals_v.at[sl],
                              sem).wait()

    for r in range(_DEPTH):
        fire(r)

    def g_body(r, _):
        fire(r + _DEPTH)
        drain(r)
        return _

    lax.fori_loop(0, _NSTREAM - _DEPTH, g_body, None)
    for r in range(_NSTREAM - _DEPTH, _NSTREAM):
        drain(r)

    biasv = bias_v[...]
    iota26 = iota * _F

    def red_body(c, _):
        acc = biasv
        b0 = c * (_L * _F)
        for f in range(_F):
            acc = acc + plsc.load_gather(vals_v, [iota26 + (b0 + f)])
        out_v[pl.ds(c * _L, _L)] = 1.0 / (1.0 + jnp.exp(-acc))
        return _

    lax.fori_loop(0, _BPW // _L, red_body, None)
    pltpu.sync_copy(out_v, out_hbm.at[pl.ds(wid * _BPW, _BPW)])


def kernel(x, table, bias):
    out = _lr_kernel(x.reshape(-1), table.reshape(-1),
                     jnp.broadcast_to(bias, (_L,)))
    return out.reshape(_B, 1)


# speedup vs baseline: 1.0974x; 1.0233x over previous
import functools

import jax
import jax.numpy as jnp
from jax import lax
from jax.experimental import pallas as pl
from jax.experimental.pallas import tpu as pltpu
from jax.experimental.pallas import tpu_sc as plsc

_B, _F = 16384, 26
_OFF = 100000
_NC, _NS, _L = 2, 16, 16
_NW = _NC * _NS
_BPW = _B // _NW
_IPW = _BPW * _F
_NCHUNK = _IPW // _L
_SLEN = 128
_NSTREAM = _IPW // _SLEN
_DEPTH = 6

_mesh = plsc.VectorSubcoreMesh(core_axis_name="c", subcore_axis_name="s")


@functools.partial(
    pl.kernel,
    out_type=jax.ShapeDtypeStruct((_B,), jnp.float32),
    mesh=_mesh,
    compiler_params=pltpu.CompilerParams(needs_layout_passes=False),
    scratch_types=[
        pltpu.VMEM((_IPW,), jnp.int32),
        pltpu.VMEM((_IPW,), jnp.float32),
        pltpu.VMEM((_BPW,), jnp.float32),
        pltpu.VMEM((_L,), jnp.float32),
        pltpu.SemaphoreType.DMA,
    ],
)
def _lr_kernel(x_hbm, table_hbm, bias_hbm, out_hbm, idx_v, vals_v, out_v,
               bias_v, sem):
    wid = lax.axis_index("s") * _NC + lax.axis_index("c")
    base = wid * _IPW

    pltpu.sync_copy(x_hbm.at[pl.ds(base, _IPW)], idx_v)
    pltpu.sync_copy(bias_hbm, bias_v)

    iota = lax.iota(jnp.int32, _L)

    def prep(r):
        for k in range(_SLEN // _L):
            sl = pl.ds((r * _SLEN // _L + k) * _L, _L)
            f = lax.rem((r * _SLEN // _L + k) * _L + iota, _F)
            idx_v[sl] = idx_v[sl] + f * _OFF

    def fire(r):
        sl = pl.ds(r * _SLEN, _SLEN)
        pltpu.async_copy(table_hbm.at[idx_v.at[sl]], vals_v.at[sl], sem)

    def drain(r):
        sl = pl.ds(r * _SLEN, _SLEN)
        pltpu.make_async_copy(table_hbm.at[idx_v.at[sl]], vals_v.at[sl],
                              sem).wait()

    for r in range(_DEPTH):
        prep(r)
        fire(r)

    def g_body(r, _):
        prep(r + _DEPTH)
        fire(r + _DEPTH)
        drain(r)
        return _

    lax.fori_loop(0, _NSTREAM - _DEPTH, g_body, None)

    biasv = bias_v[...]
    iota26 = iota * _F

    def red_body(c, _):
        acc = biasv
        b0 = c * (_L * _F)
        for f in range(_F):
            acc = acc + plsc.load_gather(vals_v, [iota26 + (b0 + f)])
        out_v[pl.ds(c * _L, _L)] = 1.0 / (1.0 + jnp.exp(-acc))
        return _

    _safe = ((_NSTREAM - _DEPTH) * _SLEN) // (_L * _F)
    lax.fori_loop(0, _safe, red_body, None)
    for r in range(_NSTREAM - _DEPTH, _NSTREAM):
        drain(r)
    lax.fori_loop(_safe, _BPW // _L, red_body, None)
    pltpu.sync_copy(out_v, out_hbm.at[pl.ds(wid * _BPW, _BPW)])


def kernel(x, table, bias):
    out = _lr_kernel(x.reshape(-1), table.reshape(-1),
                     jnp.broadcast_to(bias, (_L,)))
    return out.reshape(_B, 1)
